# baseline (speedup 1.0000x reference)
import functools

import jax
import jax.numpy as jnp
from jax import lax
from jax.experimental import pallas as pl
from jax.experimental.pallas import tpu as pltpu
from jax.experimental.pallas import tpu_sc as plsc

N_PAD = 10240
E_PAD = 327680
NC, NS = 2, 16
NW = NC * NS
EPW = E_PAD // NW
G = 64
RB = 1024
EB = 2048

_MESH = dict(core_axis_name="c", subcore_axis_name="s",
             num_cores=NC, num_subcores=NS)


def _tc_node(h, p):
    cin = h.shape[1]
    C = p["msg_mlp"]["w"].shape[1]
    mw = p["msg_mlp"]["w"]
    mb = p["msg_mlp"]["b"].reshape(1, C)
    nw = p["node_mlp"]["w"]
    nb = p["node_mlp"]["b"].reshape(1, C)
    aw = jnp.pad(p["alpha_mlp"]["w"], ((0, 0), (0, 112)))
    ab = jnp.pad(p["alpha_mlp"]["b"], (0, 112)).reshape(1, 128)
    sw = jnp.pad(p["score"]["w"].reshape(16), (0, 112)).reshape(1, 128)

    def body(h_ref, mw_ref, mb_ref, nw_ref, nb_ref, aw_ref, ab_ref, sw_ref,
             a_ref, b_ref, xt_ref):
        hb = h_ref[...]
        W = mw_ref[...]
        W1 = W[:cin]
        W2 = W[cin:]
        a_ref[:, :C] = jnp.dot(hb, W1 - W2, preferred_element_type=jnp.float32) + mb_ref[...]
        a_ref[:, C:] = jnp.tanh(
            jnp.dot(hb, aw_ref[...], preferred_element_type=jnp.float32) + ab_ref[...]) * sw_ref[...]
        b_ref[...] = jnp.dot(hb, W2, preferred_element_type=jnp.float32)
        xt_ref[...] = jnp.maximum(
            jnp.dot(hb, nw_ref[...], preferred_element_type=jnp.float32) + nb_ref[...], 0.0)

    nb_blocks = N_PAD // RB
    whole = lambda s: pl.BlockSpec(s, lambda i: (0,) * len(s))
    return pl.pallas_call(
        body,
        grid=(nb_blocks,),
        in_specs=[
            pl.BlockSpec((RB, cin), lambda i: (i, 0)),
            whole((2 * cin, C)), whole((1, C)),
            whole((cin, C)), whole((1, C)),
            whole((cin, 128)), whole((1, 128)), whole((1, 128)),
        ],
        out_specs=[
            pl.BlockSpec((RB, C + 128), lambda i: (i, 0)),
            pl.BlockSpec((RB, C), lambda i: (i, 0)),
            pl.BlockSpec((RB, C), lambda i: (i, 0)),
        ],
        out_shape=[
            jax.ShapeDtypeStruct((N_PAD, C + 128), jnp.float32),
            jax.ShapeDtypeStruct((N_PAD, C), jnp.float32),
            jax.ShapeDtypeStruct((N_PAD, C), jnp.float32),
        ],
    )(h, mw, mb, nw, nb, aw, ab, sw)


def _tc_edge_logits(msgext, p):
    CE = msgext.shape[1]
    C = CE - 128
    bw = p["beta_mlp"]["w"]
    bb = p["beta_mlp"]["b"].reshape(1, 16)
    bs = p["score"]["b"].reshape(1, 1)

    def body(m_ref, bw_ref, bb_ref, bs_ref, o_ref):
        t = jnp.tanh(jnp.dot(m_ref[:, :C], bw_ref[...],
                             preferred_element_type=jnp.float32) + bb_ref[...])
        w = jnp.sum(m_ref[:, C:C + 16] * t, axis=1, keepdims=True) + bs_ref[...]
        o_ref[...] = jnp.exp(w)

    whole = lambda s: pl.BlockSpec(s, lambda i: (0,) * len(s))
    return pl.pallas_call(
        body,
        grid=(E_PAD // EB,),
        in_specs=[
            pl.BlockSpec((EB, CE), lambda i: (i, 0)),
            whole((C, 16)), whole((1, 16)), whole((1, 1)),
        ],
        out_specs=pl.BlockSpec((EB, 1), lambda i: (i, 0)),
        out_shape=jax.ShapeDtypeStruct((E_PAD, 1), jnp.float32),
    )(msgext, bw, bb, bs)


def _tc_scale(msgext, r, C):
    HC = C // 128

    def body(m_ref, r_ref, o_ref):
        o_ref[...] = m_ref[...] * r_ref[...]

    return pl.pallas_call(
        body,
        grid=(E_PAD // EB, HC),
        in_specs=[
            pl.BlockSpec((EB, 128), lambda i, j: (i, j)),
            pl.BlockSpec((EB, 1), lambda i, j: (i, 0)),
        ],
        out_specs=pl.BlockSpec((EB, 128), lambda i, j: (i, j)),
        out_shape=jax.ShapeDtypeStruct((E_PAD, C), jnp.float32),
    )(msgext, r)


def _tc_combine(agg2, xt, p):
    C = xt.shape[1]
    wm = p["w_msg"]["w"]
    bm = p["w_msg"]["b"].reshape(1, 1)
    wn = p["w_node"]["w"]
    bn = p["w_node"]["b"].reshape(1, 1)

    def body(ag_ref, xt_ref, wm_ref, bm_ref, wn_ref, bn_ref, o_ref):
        agg = ag_ref[...]
        x_t = xt_ref[...]
        wmv = wm_ref[...]
        wnv = wn_ref[...]
        w1 = jax.nn.sigmoid(
            jnp.dot(x_t, wmv[:C], preferred_element_type=jnp.float32)
            + jnp.dot(agg, wmv[C:], preferred_element_type=jnp.float32) + bm_ref[...])
        w2 = jax.nn.sigmoid(
            jnp.dot(x_t, wnv[:C], preferred_element_type=jnp.float32)
            + jnp.dot(agg, wnv[C:], preferred_element_type=jnp.float32) + bn_ref[...])
        o_ref[...] = jnp.maximum(w1 * agg + w2 * x_t, 0.0)

    whole = lambda s: pl.BlockSpec(s, lambda i: (0,) * len(s))
    return pl.pallas_call(
        body,
        grid=(N_PAD // RB,),
        in_specs=[
            pl.BlockSpec((RB, C), lambda i: (i, 0)),
            pl.BlockSpec((RB, C), lambda i: (i, 0)),
            whole((2 * C, 1)), whole((1, 1)),
            whole((2 * C, 1)), whole((1, 1)),
        ],
        out_specs=pl.BlockSpec((RB, C), lambda i: (i, 0)),
        out_shape=jax.ShapeDtypeStruct((N_PAD, C), jnp.float32),
    )(agg2, xt, wm, bm, wn, bn)


def _tc_pool(h, batch2d, p):
    C = h.shape[1]
    pw = p["w"]
    pb = p["b"].reshape(1, 1)

    def body(h_ref, b_ref, pw_ref, pb_ref, o_ref):
        hh = h_ref[...]
        gate = jnp.dot(hh, pw_ref[...], preferred_element_type=jnp.float32) + pb_ref[...]
        gid = lax.broadcasted_iota(jnp.int32, (1, G), 1)
        mask = b_ref[...] == gid
        logits = jnp.where(mask, gate, -1e30)
        m = jnp.max(logits, axis=0, keepdims=True)
        mm = jnp.where(m > -1e29, m, 0.0)
        e = jnp.where(mask, jnp.exp(logits - mm), 0.0)
        s = jnp.sum(e, axis=0, keepdims=True)
        wgt = e / (s + 1e-16)
        o_ref[...] = lax.dot_general(wgt, hh, (((0,), (0,)), ((), ())),
                                     preferred_element_type=jnp.float32)

    whole = lambda s: pl.BlockSpec(s, lambda i: (0,) * len(s))
    return pl.pallas_call(
        body,
        grid=(1,),
        in_specs=[
            whole((N_PAD, C)), whole((N_PAD, 1)), whole((C, 1)), whole((1, 1)),
        ],
        out_specs=whole((G, C)),
        out_shape=jax.ShapeDtypeStruct((G, C), jnp.float32),
    )(h, batch2d, pw, pb)


def _sc_gather_msg(Aext, B, dst2, src2, C):
    CE = C + 128
    CH = 128 if C == 256 else 256
    K = CH // 128
    CHUNKS = EPW // CH
    mesh = plsc.VectorSubcoreMesh(**_MESH)

    @functools.partial(
        pl.kernel,
        out_type=jax.ShapeDtypeStruct((E_PAD, CE), jnp.float32),
        mesh=mesh,
        scratch_types=[
            pltpu.VMEM((K, 128), jnp.int32),
            pltpu.VMEM((K, 128), jnp.int32),
            pltpu.VMEM((CH, CE), jnp.float32),
            pltpu.VMEM((CH, C), jnp.float32),
            pltpu.SemaphoreType.DMA,
        ],
    )
    def k(a_h, b_h, dst_h, src_h, msg_o, idxd, idxs, ai, bj, gsem):
        wid = lax.axis_index("s") * NC + lax.axis_index("c")

        def chunk(i, carry):
            row0 = wid * (EPW // 128) + i * K
            e0 = wid * EPW + i * CH
            pltpu.sync_copy(dst_h.at[pl.ds(row0, K)], idxd)
            pltpu.sync_copy(src_h.at[pl.ds(row0, K)], idxs)
            cps = []
            for j in range(K):
                cps.append(pltpu.async_copy(
                    a_h.at[idxd.at[j]], ai.at[pl.ds(j * 128, 128)], gsem))
                cps.append(pltpu.async_copy(
                    b_h.at[idxs.at[j]], bj.at[pl.ds(j * 128, 128)], gsem))
            for cp in cps:
                cp.wait()

            def row(rr, c2):
                for q in range(C // 16):
                    sl = pl.ds(q * 16, 16)
                    ai[rr, sl] = jnp.maximum(ai[rr, sl] + bj[rr, sl], 0.0)
                return c2

            lax.fori_loop(0, CH, row, 0)
            pltpu.sync_copy(ai, msg_o.at[pl.ds(e0, CH)])
            return carry

        lax.fori_loop(0, CHUNKS, chunk, 0)

    return k(Aext, B, dst2, src2)


def _sc_denom(ew, src2):
    CH = 1024
    K = CH // 128
    CHUNKS = EPW // CH
    SL = N_PAD // NS
    mesh = plsc.VectorSubcoreMesh(**_MESH)

    @functools.partial(
        pl.kernel,
        out_type=[jax.ShapeDtypeStruct((N_PAD,), jnp.float32),
                  jax.ShapeDtypeStruct((N_PAD,), jnp.float32)],
        mesh=mesh,
        scratch_types=[
            pltpu.VMEM((CH,), jnp.float32),
            pltpu.VMEM((K, 128), jnp.int32),
            pltpu.VMEM((SL,), jnp.float32),
            pltpu.VMEM_SHARED((N_PAD,), jnp.float32),
            pltpu.SemaphoreType.DMA,
        ],
    )
    def k(ew_h, src_h, s0_o, s1_o, ewv, idxs, zb, ssh, sem):
        cid = lax.axis_index("c")
        sid = lax.axis_index("s")
        wid = sid * NC + cid

        def z(i, c2):
            zb[pl.ds(i * 16, 16)] = jnp.zeros((16,), jnp.float32)
            return c2

        lax.fori_loop(0, SL // 16, z, 0)
        pltpu.sync_copy(zb, ssh.at[pl.ds(sid * SL, SL)])
        plsc.subcore_barrier()

        def chunk(i, carry):
            row0 = wid * (EPW // 128) + i * K
            e0 = wid * EPW + i * CH
            pltpu.sync_copy(ew_h.at[pl.ds(e0, CH)], ewv)
            pltpu.sync_copy(src_h.at[pl.ds(row0, K)], idxs)
            cps = [pltpu.async_copy(ewv.at[pl.ds(j * 128, 128)],
                                    ssh.at[idxs.at[j]], sem, add=True)
                   for j in range(K)]
            for cp in cps:
                cp.wait()
            return carry

        lax.fori_loop(0, CHUNKS, chunk, 0)
        plsc.subcore_barrier()

        @pl.when(cid == 0)
        def _():
            pltpu.sync_copy(ssh.at[pl.ds(sid * SL, SL)], s0_o.at[pl.ds(sid * SL, SL)])

        @pl.when(cid == 1)
        def _():
            pltpu.sync_copy(ssh.at[pl.ds(sid * SL, SL)], s1_o.at[pl.ds(sid * SL, SL)])

    return k(ew, src2)


def _sc_normalize(ew, src2, s0, s1):
    CH = 2048
    K = CH // 128
    CHUNKS = EPW // CH
    mesh = plsc.VectorSubcoreMesh(**_MESH)

    @functools.partial(
        pl.kernel,
        out_type=jax.ShapeDtypeStruct((E_PAD,), jnp.float32),
        mesh=mesh,
        scratch_types=[
            pltpu.VMEM((K, 128), jnp.int32),
            pltpu.VMEM((CH,), jnp.float32),
            pltpu.VMEM((CH,), jnp.float32),
            pltpu.VMEM((CH,), jnp.float32),
            pltpu.SemaphoreType.DMA,
        ],
    )
    def k(ew_h, src_h, s0_h, s1_h, r_o, sidx, g0, g1, ewv, sem):
        wid = lax.axis_index("s") * NC + lax.axis_index("c")

        def chunk(i, carry):
            row0 = wid * (EPW // 128) + i * K
            e0 = wid * EPW + i * CH
            pltpu.sync_copy(ew_h.at[pl.ds(e0, CH)], ewv)
            pltpu.sync_copy(src_h.at[pl.ds(row0, K)], sidx)
            cps = []
            for j in range(K):
                cps.append(pltpu.async_copy(s0_h.at[sidx.at[j]],
                                            g0.at[pl.ds(j * 128, 128)], sem))
                cps.append(pltpu.async_copy(s1_h.at[sidx.at[j]],
                                            g1.at[pl.ds(j * 128, 128)], sem))
            for cp in cps:
                cp.wait()

            def lane(l, c2):
                sl = pl.ds(l * 16, 16)
                ewv[sl] = ewv[sl] / (g0[sl] + g1[sl] + 1e-16)
                return c2

            lax.fori_loop(0, CH // 16, lane, 0)
            pltpu.sync_copy(ewv, r_o.at[pl.ds(e0, CH)])
            return carry

        lax.fori_loop(0, CHUNKS, chunk, 0)

    return k(ew, src2, s0, s1)


def _sc_aggregate(msgext, r, zrs, dst2, C):
    CH = 512
    K = CH // 128
    NH = N_PAD // NC
    TRASH = NH
    EPW2 = E_PAD // NS
    CHUNKS = EPW2 // CH
    DR = NH // NS
    HC = C // 128
    CE = C + 128
    mesh = plsc.VectorSubcoreMesh(**_MESH)

    @functools.partial(
        pl.kernel,
        out_type=jax.ShapeDtypeStruct((N_PAD, C), jnp.float32),
        mesh=mesh,
        scratch_types=[
            pltpu.VMEM((CH, 128), jnp.float32),
            pltpu.VMEM((CH,), jnp.float32),
            pltpu.VMEM((K, 128), jnp.int32),
            pltpu.VMEM((K, 128), jnp.int32),
            pltpu.VMEM_SHARED((NH + 8, 128), jnp.float32),
            pltpu.SemaphoreType.DMA,
        ],
    )
    def k(m_h, r_h, z_h, dst_h, agg_o, cb, rv, idxd, idxr, ash, sem):
        cid = lax.axis_index("c")
        sid = lax.axis_index("s")
        base = cid * NH

        for hc in range(HC):
            pltpu.sync_copy(z_h.at[pl.ds(0, DR)], ash.at[pl.ds(sid * DR, DR)])

            @pl.when(sid == 0)
            def _():
                pltpu.sync_copy(z_h.at[pl.ds(0, 8)], ash.at[pl.ds(NH, 8)])

            plsc.subcore_barrier()

            def chunk(i, carry):
                row0 = sid * (EPW2 // 128) + i * K
                e0 = sid * EPW2 + i * CH
                pltpu.sync_copy(m_h.at[pl.ds(e0, CH), pl.ds(hc * 128, 128)], cb)
                pltpu.sync_copy(r_h.at[pl.ds(e0, CH)], rv)
                pltpu.sync_copy(dst_h.at[pl.ds(row0, K)], idxd)
                for j in range(K):
                    for l in range(8):
                        sl = pl.ds(l * 16, 16)
                        d = idxd[j, sl]
                        idxr[j, sl] = jnp.where((d >= base) & (d < base + NH),
                                                d - base, TRASH + (d & 7))

                def rgrp(g, c2):
                    rvec = rv[pl.ds(g * 16, 16)]
                    for t in range(16):
                        s = rvec[t]
                        for q in range(8):
                            sl = pl.ds(q * 16, 16)
                            cb[g * 16 + t, sl] = cb[g * 16 + t, sl] * s
                    return c2

                lax.fori_loop(0, CH // 16, rgrp, 0)
                cps = [pltpu.async_copy(cb.at[pl.ds(j * 128, 128)],
                                        ash.at[idxr.at[j]], sem, add=True)
                       for j in range(K)]
                for cp in cps:
                    cp.wait()
                return carry

            lax.fori_loop(0, CHUNKS, chunk, 0)
            plsc.subcore_barrier()
            if HC == 1:
                pltpu.sync_copy(ash.at[pl.ds(sid * DR, DR)],
                                agg_o.at[pl.ds(base + sid * DR, DR)])
            else:
                pltpu.sync_copy(ash.at[pl.ds(sid * DR, DR)],
                                agg_o.at[pl.ds(base + sid * DR, DR),
                                         pl.ds(hc * 128, 128)])
            if hc + 1 < HC:
                plsc.subcore_barrier()

    return k(msgext, r, zrs, dst2)


def _layer(p, h, src2, dst2, srcf, zrs):
    C = p["msg_mlp"]["w"].shape[1]
    Aext, B, xt = _tc_node(h, p)
    msgext = _sc_gather_msg(Aext, B, dst2, src2, C)
    ew = _tc_edge_logits(msgext, p)
    ewf = ew.reshape(E_PAD)
    s0, s1 = _sc_denom(ewf, src2)
    r = _sc_normalize(ewf, src2, s0, s1)
    agg = _sc_aggregate(msgext, r, zrs, dst2, C)
    return _tc_combine(agg, xt, p)


def kernel(x, edge_index, batch, params):
    x = x.astype(jnp.float32)
    src = edge_index[0].astype(jnp.int32)
    dst = edge_index[1].astype(jnp.int32)
    N = x.shape[0]
    E = src.shape[0]
    x_pad = jnp.pad(x, ((0, N_PAD - N), (0, 0)))
    srcf = jnp.pad(src, (0, E_PAD - E), constant_values=N)
    dstf = jnp.pad(dst, (0, E_PAD - E), constant_values=N)
    src2 = srcf.reshape(E_PAD // 128, 128)
    dst2 = dstf.reshape(E_PAD // 128, 128)
    b2 = jnp.pad(batch.astype(jnp.int32), (0, N_PAD - N),
                 constant_values=G).reshape(N_PAD, 1)
    zrs = jnp.zeros((N_PAD // NC // NS, 128), jnp.float32)

    h = _layer(params["conv1"], x_pad, src2, dst2, srcf, zrs)
    h = _layer(params["conv2"], h, src2, dst2, srcf, zrs)
    x1 = _tc_pool(h, b2, params["pool1"])
    h = _layer(params["conv3"], h, src2, dst2, srcf, zrs)
    h = _layer(params["conv4"], h, src2, dst2, srcf, zrs)
    x2 = _tc_pool(h, b2, params["pool2"])
    return jnp.concatenate([x1, x2], axis=1)


# speedup vs baseline: 1.0543x; 1.0543x over previous
import functools

import jax
import jax.numpy as jnp
from jax import lax
from jax.experimental import pallas as pl
from jax.experimental.pallas import tpu as pltpu
from jax.experimental.pallas import tpu_sc as plsc

N_PAD = 10240
E_PAD = 327680
NC, NS = 2, 16
NW = NC * NS
EPW = E_PAD // NW
G = 64
RB = 1024
EB = 2048

_MESH = dict(core_axis_name="c", subcore_axis_name="s",
             num_cores=NC, num_subcores=NS)


def _tc_node(h, p):
    cin = h.shape[1]
    C = p["msg_mlp"]["w"].shape[1]
    mw = p["msg_mlp"]["w"]
    mb = p["msg_mlp"]["b"].reshape(1, C)
    nw = p["node_mlp"]["w"]
    nb = p["node_mlp"]["b"].reshape(1, C)
    aw = jnp.pad(p["alpha_mlp"]["w"], ((0, 0), (0, 112)))
    ab = jnp.pad(p["alpha_mlp"]["b"], (0, 112)).reshape(1, 128)
    sw = jnp.pad(p["score"]["w"].reshape(16), (0, 112)).reshape(1, 128)

    def body(h_ref, mw_ref, mb_ref, nw_ref, nb_ref, aw_ref, ab_ref, sw_ref,
             a_ref, b_ref, xt_ref):
        hb = h_ref[...]
        W = mw_ref[...]
        W1 = W[:cin]
        W2 = W[cin:]
        a_ref[:, :C] = jnp.dot(hb, W1 - W2, preferred_element_type=jnp.float32) + mb_ref[...]
        a_ref[:, C:] = jnp.tanh(
            jnp.dot(hb, aw_ref[...], preferred_element_type=jnp.float32) + ab_ref[...]) * sw_ref[...]
        b_ref[...] = jnp.dot(hb, W2, preferred_element_type=jnp.float32)
        xt_ref[...] = jnp.maximum(
            jnp.dot(hb, nw_ref[...], preferred_element_type=jnp.float32) + nb_ref[...], 0.0)

    nb_blocks = N_PAD // RB
    whole = lambda s: pl.BlockSpec(s, lambda i: (0,) * len(s))
    return pl.pallas_call(
        body,
        grid=(nb_blocks,),
        in_specs=[
            pl.BlockSpec((RB, cin), lambda i: (i, 0)),
            whole((2 * cin, C)), whole((1, C)),
            whole((cin, C)), whole((1, C)),
            whole((cin, 128)), whole((1, 128)), whole((1, 128)),
        ],
        out_specs=[
            pl.BlockSpec((RB, C + 128), lambda i: (i, 0)),
            pl.BlockSpec((RB, C), lambda i: (i, 0)),
            pl.BlockSpec((RB, C), lambda i: (i, 0)),
        ],
        out_shape=[
            jax.ShapeDtypeStruct((N_PAD, C + 128), jnp.float32),
            jax.ShapeDtypeStruct((N_PAD, C), jnp.float32),
            jax.ShapeDtypeStruct((N_PAD, C), jnp.float32),
        ],
    )(h, mw, mb, nw, nb, aw, ab, sw)


def _tc_edge_logits(msgext, p):
    CE = msgext.shape[1]
    C = CE - 128
    bw = p["beta_mlp"]["w"]
    bb = p["beta_mlp"]["b"].reshape(1, 16)
    bs = p["score"]["b"].reshape(1, 1)

    def body(m_ref, bw_ref, bb_ref, bs_ref, o_ref):
        t = jnp.tanh(jnp.dot(m_ref[:, :C], bw_ref[...],
                             preferred_element_type=jnp.float32) + bb_ref[...])
        w = jnp.sum(m_ref[:, C:C + 16] * t, axis=1, keepdims=True) + bs_ref[...]
        o_ref[...] = jnp.exp(w)

    whole = lambda s: pl.BlockSpec(s, lambda i: (0,) * len(s))
    return pl.pallas_call(
        body,
        grid=(E_PAD // EB,),
        in_specs=[
            pl.BlockSpec((EB, CE), lambda i: (i, 0)),
            whole((C, 16)), whole((1, 16)), whole((1, 1)),
        ],
        out_specs=pl.BlockSpec((EB, 1), lambda i: (i, 0)),
        out_shape=jax.ShapeDtypeStruct((E_PAD, 1), jnp.float32),
    )(msgext, bw, bb, bs)


def _tc_scale(msgext, r, C):
    HC = C // 128

    def body(m_ref, r_ref, o_ref):
        o_ref[...] = m_ref[...] * r_ref[...]

    return pl.pallas_call(
        body,
        grid=(E_PAD // EB, HC),
        in_specs=[
            pl.BlockSpec((EB, 128), lambda i, j: (i, j)),
            pl.BlockSpec((EB, 1), lambda i, j: (i, 0)),
        ],
        out_specs=pl.BlockSpec((EB, 128), lambda i, j: (i, j)),
        out_shape=jax.ShapeDtypeStruct((E_PAD, C), jnp.float32),
    )(msgext, r)


def _tc_combine(agg2, xt, p):
    C = xt.shape[1]
    wm = p["w_msg"]["w"]
    bm = p["w_msg"]["b"].reshape(1, 1)
    wn = p["w_node"]["w"]
    bn = p["w_node"]["b"].reshape(1, 1)

    def body(ag_ref, xt_ref, wm_ref, bm_ref, wn_ref, bn_ref, o_ref):
        agg = ag_ref[...]
        x_t = xt_ref[...]
        wmv = wm_ref[...]
        wnv = wn_ref[...]
        w1 = jax.nn.sigmoid(
            jnp.dot(x_t, wmv[:C], preferred_element_type=jnp.float32)
            + jnp.dot(agg, wmv[C:], preferred_element_type=jnp.float32) + bm_ref[...])
        w2 = jax.nn.sigmoid(
            jnp.dot(x_t, wnv[:C], preferred_element_type=jnp.float32)
            + jnp.dot(agg, wnv[C:], preferred_element_type=jnp.float32) + bn_ref[...])
        o_ref[...] = jnp.maximum(w1 * agg + w2 * x_t, 0.0)

    whole = lambda s: pl.BlockSpec(s, lambda i: (0,) * len(s))
    return pl.pallas_call(
        body,
        grid=(N_PAD // RB,),
        in_specs=[
            pl.BlockSpec((RB, C), lambda i: (i, 0)),
            pl.BlockSpec((RB, C), lambda i: (i, 0)),
            whole((2 * C, 1)), whole((1, 1)),
            whole((2 * C, 1)), whole((1, 1)),
        ],
        out_specs=pl.BlockSpec((RB, C), lambda i: (i, 0)),
        out_shape=jax.ShapeDtypeStruct((N_PAD, C), jnp.float32),
    )(agg2, xt, wm, bm, wn, bn)


def _tc_combine_node(agg2, xt, p, pn):
    C = xt.shape[1]
    C2 = pn["msg_mlp"]["w"].shape[1]
    wm = p["w_msg"]["w"]
    bm = p["w_msg"]["b"].reshape(1, 1)
    wn = p["w_node"]["w"]
    bn = p["w_node"]["b"].reshape(1, 1)
    mw = pn["msg_mlp"]["w"]
    mb = pn["msg_mlp"]["b"].reshape(1, C2)
    nw2 = pn["node_mlp"]["w"]
    nb2 = pn["node_mlp"]["b"].reshape(1, C2)
    aw = jnp.pad(pn["alpha_mlp"]["w"], ((0, 0), (0, 112)))
    ab = jnp.pad(pn["alpha_mlp"]["b"], (0, 112)).reshape(1, 128)
    sw = jnp.pad(pn["score"]["w"].reshape(16), (0, 112)).reshape(1, 128)

    def body(ag_ref, xt_ref, wm_ref, bm_ref, wn_ref, bn_ref,
             mw_ref, mb_ref, nw_ref, nb_ref, aw_ref, ab_ref, sw_ref,
             h_ref, a_ref, b_ref, x2_ref):
        agg = ag_ref[...]
        x_t = xt_ref[...]
        wmv = wm_ref[...]
        wnv = wn_ref[...]
        w1 = jax.nn.sigmoid(
            jnp.dot(x_t, wmv[:C], preferred_element_type=jnp.float32)
            + jnp.dot(agg, wmv[C:], preferred_element_type=jnp.float32) + bm_ref[...])
        w2 = jax.nn.sigmoid(
            jnp.dot(x_t, wnv[:C], preferred_element_type=jnp.float32)
            + jnp.dot(agg, wnv[C:], preferred_element_type=jnp.float32) + bn_ref[...])
        hb = jnp.maximum(w1 * agg + w2 * x_t, 0.0)
        h_ref[...] = hb
        W = mw_ref[...]
        W1 = W[:C]
        W2 = W[C:]
        a_ref[:, :C2] = jnp.dot(hb, W1 - W2, preferred_element_type=jnp.float32) + mb_ref[...]
        a_ref[:, C2:] = jnp.tanh(
            jnp.dot(hb, aw_ref[...], preferred_element_type=jnp.float32) + ab_ref[...]) * sw_ref[...]
        b_ref[...] = jnp.dot(hb, W2, preferred_element_type=jnp.float32)
        x2_ref[...] = jnp.maximum(
            jnp.dot(hb, nw_ref[...], preferred_element_type=jnp.float32) + nb_ref[...], 0.0)

    whole = lambda s: pl.BlockSpec(s, lambda i: (0,) * len(s))
    return pl.pallas_call(
        body,
        grid=(N_PAD // RB,),
        in_specs=[
            pl.BlockSpec((RB, C), lambda i: (i, 0)),
            pl.BlockSpec((RB, C), lambda i: (i, 0)),
            whole((2 * C, 1)), whole((1, 1)),
            whole((2 * C, 1)), whole((1, 1)),
            whole((2 * C, C2)), whole((1, C2)),
            whole((C, C2)), whole((1, C2)),
            whole((C, 128)), whole((1, 128)), whole((1, 128)),
        ],
        out_specs=[
            pl.BlockSpec((RB, C), lambda i: (i, 0)),
            pl.BlockSpec((RB, C2 + 128), lambda i: (i, 0)),
            pl.BlockSpec((RB, C2), lambda i: (i, 0)),
            pl.BlockSpec((RB, C2), lambda i: (i, 0)),
        ],
        out_shape=[
            jax.ShapeDtypeStruct((N_PAD, C), jnp.float32),
            jax.ShapeDtypeStruct((N_PAD, C2 + 128), jnp.float32),
            jax.ShapeDtypeStruct((N_PAD, C2), jnp.float32),
            jax.ShapeDtypeStruct((N_PAD, C2), jnp.float32),
        ],
    )(agg2, xt, wm, bm, wn, bn, mw, mb, nw2, nb2, aw, ab, sw)


def _tc_pool(h, batch2d, p):
    C = h.shape[1]
    pw = p["w"]
    pb = p["b"].reshape(1, 1)

    def body(h_ref, b_ref, pw_ref, pb_ref, o_ref):
        hh = h_ref[...]
        gate = jnp.dot(hh, pw_ref[...], preferred_element_type=jnp.float32) + pb_ref[...]
        gid = lax.broadcasted_iota(jnp.int32, (1, G), 1)
        mask = b_ref[...] == gid
        logits = jnp.where(mask, gate, -1e30)
        m = jnp.max(logits, axis=0, keepdims=True)
        mm = jnp.where(m > -1e29, m, 0.0)
        e = jnp.where(mask, jnp.exp(logits - mm), 0.0)
        s = jnp.sum(e, axis=0, keepdims=True)
        wgt = e / (s + 1e-16)
        o_ref[...] = lax.dot_general(wgt, hh, (((0,), (0,)), ((), ())),
                                     preferred_element_type=jnp.float32)

    whole = lambda s: pl.BlockSpec(s, lambda i: (0,) * len(s))
    return pl.pallas_call(
        body,
        grid=(1,),
        in_specs=[
            whole((N_PAD, C)), whole((N_PAD, 1)), whole((C, 1)), whole((1, 1)),
        ],
        out_specs=whole((G, C)),
        out_shape=jax.ShapeDtypeStruct((G, C), jnp.float32),
    )(h, batch2d, pw, pb)


def _sc_gather_msg(Aext, B, dst2, src2, C):
    CE = C + 128
    CH = 128
    CHUNKS = EPW // CH
    NBUF = 1 if C == 256 else 2
    mesh = plsc.VectorSubcoreMesh(**_MESH)

    scratch = []
    for _ in range(NBUF):
        scratch += [
            pltpu.VMEM((1, 128), jnp.int32),
            pltpu.VMEM((1, 128), jnp.int32),
            pltpu.VMEM((CH, CE), jnp.float32),
            pltpu.VMEM((CH, C), jnp.float32),
            pltpu.SemaphoreType.DMA,
            pltpu.SemaphoreType.DMA,
        ]

    @functools.partial(
        pl.kernel,
        out_type=jax.ShapeDtypeStruct((E_PAD, CE), jnp.float32),
        mesh=mesh,
        scratch_types=scratch,
    )
    def k(a_h, b_h, dst_h, src_h, msg_o, *bufs):
        wid = lax.axis_index("s") * NC + lax.axis_index("c")

        def buf(pp):
            return bufs[6 * pp:6 * pp + 6]

        def fire(i, pp):
            idxd, idxs, ai, bj, gsem, _ = buf(pp)
            row0 = wid * (EPW // 128) + i
            pltpu.sync_copy(dst_h.at[pl.ds(row0, 1)], idxd)
            pltpu.sync_copy(src_h.at[pl.ds(row0, 1)], idxs)
            pltpu.async_copy(a_h.at[idxd.at[0]], ai, gsem)
            pltpu.async_copy(b_h.at[idxs.at[0]], bj, gsem)

        def wait_g(pp):
            idxd, idxs, ai, bj, gsem, _ = buf(pp)
            pltpu.make_async_copy(a_h.at[pl.ds(0, CH)], ai, gsem).wait()
            pltpu.make_async_copy(b_h.at[pl.ds(0, CH)], bj, gsem).wait()

        def compute(pp):
            ai, bj = buf(pp)[2], buf(pp)[3]

            def row(rr, c2):
                for q in range(C // 16):
                    sl = pl.ds(q * 16, 16)
                    ai[rr, sl] = jnp.maximum(ai[rr, sl] + bj[rr, sl], 0.0)
                return c2

            lax.fori_loop(0, CH, row, 0)

        def fire_wb(i, pp):
            ai, wsem = buf(pp)[2], buf(pp)[5]
            e0 = wid * EPW + i * CH
            pltpu.async_copy(ai, msg_o.at[pl.ds(e0, CH)], wsem)

        def wait_wb(pp):
            ai, wsem = buf(pp)[2], buf(pp)[5]
            pltpu.make_async_copy(ai, msg_o.at[pl.ds(0, CH)], wsem).wait()

        if NBUF == 1:
            def chunk(i, carry):
                fire(i, 0)
                wait_g(0)
                compute(0)
                fire_wb(i, 0)
                wait_wb(0)
                return carry

            lax.fori_loop(0, CHUNKS, chunk, 0)
        else:
            fire(0, 0)

            def pair(g, carry):
                @pl.when(g >= 1)
                def _():
                    wait_wb(1)

                fire(2 * g + 1, 1)
                wait_g(0)
                compute(0)
                fire_wb(2 * g, 0)
                wait_g(1)
                compute(1)
                fire_wb(2 * g + 1, 1)

                @pl.when(g + 1 < CHUNKS // 2)
                def _():
                    wait_wb(0)
                    fire(2 * g + 2, 0)

                return carry

            lax.fori_loop(0, CHUNKS // 2, pair, 0)
            wait_wb(0)
            wait_wb(1)

    return k(Aext, B, dst2, src2)


def _sc_denom(ew, src2):
    CH = 1024
    K = CH // 128
    CHUNKS = EPW // CH
    SL = N_PAD // NS
    mesh = plsc.VectorSubcoreMesh(**_MESH)

    @functools.partial(
        pl.kernel,
        out_type=[jax.ShapeDtypeStruct((N_PAD,), jnp.float32),
                  jax.ShapeDtypeStruct((N_PAD,), jnp.float32)],
        mesh=mesh,
        scratch_types=[
            pltpu.VMEM((CH,), jnp.float32),
            pltpu.VMEM((K, 128), jnp.int32),
            pltpu.VMEM((SL,), jnp.float32),
            pltpu.VMEM_SHARED((N_PAD,), jnp.float32),
            pltpu.SemaphoreType.DMA,
        ],
    )
    def k(ew_h, src_h, s0_o, s1_o, ewv, idxs, zb, ssh, sem):
        cid = lax.axis_index("c")
        sid = lax.axis_index("s")
        wid = sid * NC + cid

        def z(i, c2):
            zb[pl.ds(i * 16, 16)] = jnp.zeros((16,), jnp.float32)
            return c2

        lax.fori_loop(0, SL // 16, z, 0)
        pltpu.sync_copy(zb, ssh.at[pl.ds(sid * SL, SL)])
        plsc.subcore_barrier()

        def chunk(i, carry):
            row0 = wid * (EPW // 128) + i * K
            e0 = wid * EPW + i * CH
            pltpu.sync_copy(ew_h.at[pl.ds(e0, CH)], ewv)
            pltpu.sync_copy(src_h.at[pl.ds(row0, K)], idxs)
            cps = [pltpu.async_copy(ewv.at[pl.ds(j * 128, 128)],
                                    ssh.at[idxs.at[j]], sem, add=True)
                   for j in range(K)]
            for cp in cps:
                cp.wait()
            return carry

        lax.fori_loop(0, CHUNKS, chunk, 0)
        plsc.subcore_barrier()

        @pl.when(cid == 0)
        def _():
            pltpu.sync_copy(ssh.at[pl.ds(sid * SL, SL)], s0_o.at[pl.ds(sid * SL, SL)])

        @pl.when(cid == 1)
        def _():
            pltpu.sync_copy(ssh.at[pl.ds(sid * SL, SL)], s1_o.at[pl.ds(sid * SL, SL)])

    return k(ew, src2)


def _sc_normalize(ew, src2, s0, s1):
    CH = 2048
    K = CH // 128
    CHUNKS = EPW // CH
    mesh = plsc.VectorSubcoreMesh(**_MESH)

    @functools.partial(
        pl.kernel,
        out_type=jax.ShapeDtypeStruct((E_PAD,), jnp.float32),
        mesh=mesh,
        scratch_types=[
            pltpu.VMEM((K, 128), jnp.int32),
            pltpu.VMEM((CH,), jnp.float32),
            pltpu.VMEM((CH,), jnp.float32),
            pltpu.VMEM((CH,), jnp.float32),
            pltpu.SemaphoreType.DMA,
        ],
    )
    def k(ew_h, src_h, s0_h, s1_h, r_o, sidx, g0, g1, ewv, sem):
        wid = lax.axis_index("s") * NC + lax.axis_index("c")

        def chunk(i, carry):
            row0 = wid * (EPW // 128) + i * K
            e0 = wid * EPW + i * CH
            pltpu.sync_copy(ew_h.at[pl.ds(e0, CH)], ewv)
            pltpu.sync_copy(src_h.at[pl.ds(row0, K)], sidx)
            cps = []
            for j in range(K):
                cps.append(pltpu.async_copy(s0_h.at[sidx.at[j]],
                                            g0.at[pl.ds(j * 128, 128)], sem))
                cps.append(pltpu.async_copy(s1_h.at[sidx.at[j]],
                                            g1.at[pl.ds(j * 128, 128)], sem))
            for cp in cps:
                cp.wait()

            def lane(l, c2):
                sl = pl.ds(l * 16, 16)
                ewv[sl] = ewv[sl] / (g0[sl] + g1[sl] + 1e-16)
                return c2

            lax.fori_loop(0, CH // 16, lane, 0)
            pltpu.sync_copy(ewv, r_o.at[pl.ds(e0, CH)])
            return carry

        lax.fori_loop(0, CHUNKS, chunk, 0)

    return k(ew, src2, s0, s1)


def _sc_aggregate(msgext, r, zrs, dst2, C):
    CH = 512
    K = CH // 128
    NH = N_PAD // NC
    TRASH = NH
    EPW2 = E_PAD // NS
    CHUNKS = EPW2 // CH
    DR = NH // NS
    HC = C // 128
    CE = C + 128
    mesh = plsc.VectorSubcoreMesh(**_MESH)

    @functools.partial(
        pl.kernel,
        out_type=jax.ShapeDtypeStruct((N_PAD, C), jnp.float32),
        mesh=mesh,
        scratch_types=[
            pltpu.VMEM((CH, 128), jnp.float32),
            pltpu.VMEM((CH,), jnp.float32),
            pltpu.VMEM((K, 128), jnp.int32),
            pltpu.VMEM((K, 128), jnp.int32),
            pltpu.VMEM_SHARED((NH + 8, 128), jnp.float32),
            pltpu.SemaphoreType.DMA,
        ],
    )
    def k(m_h, r_h, z_h, dst_h, agg_o, cb, rv, idxd, idxr, ash, sem):
        cid = lax.axis_index("c")
        sid = lax.axis_index("s")
        base = cid * NH

        for hc in range(HC):
            pltpu.sync_copy(z_h.at[pl.ds(0, DR)], ash.at[pl.ds(sid * DR, DR)])

            @pl.when(sid == 0)
            def _():
                pltpu.sync_copy(z_h.at[pl.ds(0, 8)], ash.at[pl.ds(NH, 8)])

            plsc.subcore_barrier()

            def chunk(i, carry):
                row0 = sid * (EPW2 // 128) + i * K
                e0 = sid * EPW2 + i * CH
                pltpu.sync_copy(m_h.at[pl.ds(e0, CH), pl.ds(hc * 128, 128)], cb)
                pltpu.sync_copy(r_h.at[pl.ds(e0, CH)], rv)
                pltpu.sync_copy(dst_h.at[pl.ds(row0, K)], idxd)
                for j in range(K):
                    for l in range(8):
                        sl = pl.ds(l * 16, 16)
                        d = idxd[j, sl]
                        idxr[j, sl] = jnp.where((d >= base) & (d < base + NH),
                                                d - base, TRASH + (d & 7))

                def rgrp(g, c2):
                    rvec = rv[pl.ds(g * 16, 16)]
                    for t in range(16):
                        s = rvec[t]
                        for q in range(8):
                            sl = pl.ds(q * 16, 16)
                            cb[g * 16 + t, sl] = cb[g * 16 + t, sl] * s
                    return c2

                lax.fori_loop(0, CH // 16, rgrp, 0)
                cps = [pltpu.async_copy(cb.at[pl.ds(j * 128, 128)],
                                        ash.at[idxr.at[j]], sem, add=True)
                       for j in range(K)]
                for cp in cps:
                    cp.wait()
                return carry

            lax.fori_loop(0, CHUNKS, chunk, 0)
            plsc.subcore_barrier()
            if HC == 1:
                pltpu.sync_copy(ash.at[pl.ds(sid * DR, DR)],
                                agg_o.at[pl.ds(base + sid * DR, DR)])
            else:
                pltpu.sync_copy(ash.at[pl.ds(sid * DR, DR)],
                                agg_o.at[pl.ds(base + sid * DR, DR),
                                         pl.ds(hc * 128, 128)])
            if hc + 1 < HC:
                plsc.subcore_barrier()

    return k(msgext, r, zrs, dst2)


def _edge_phase(p, Aext, B, src2, dst2, zrs):
    C = p["msg_mlp"]["w"].shape[1]
    msgext = _sc_gather_msg(Aext, B, dst2, src2, C)
    ew = _tc_edge_logits(msgext, p)
    ewf = ew.reshape(E_PAD)
    s0, s1 = _sc_denom(ewf, src2)
    r = _sc_normalize(ewf, src2, s0, s1)
    return _sc_aggregate(msgext, r, zrs, dst2, C)


def kernel(x, edge_index, batch, params):
    x = x.astype(jnp.float32)
    src = edge_index[0].astype(jnp.int32)
    dst = edge_index[1].astype(jnp.int32)
    N = x.shape[0]
    E = src.shape[0]
    x_pad = jnp.pad(x, ((0, N_PAD - N), (0, 0)))
    srcf = jnp.pad(src, (0, E_PAD - E), constant_values=N)
    dstf = jnp.pad(dst, (0, E_PAD - E), constant_values=N)
    src2 = srcf.reshape(E_PAD // 128, 128)
    dst2 = dstf.reshape(E_PAD // 128, 128)
    b2 = jnp.pad(batch.astype(jnp.int32), (0, N_PAD - N),
                 constant_values=G).reshape(N_PAD, 1)
    zrs = jnp.zeros((N_PAD // NC // NS, 128), jnp.float32)

    p1, p2 = params["conv1"], params["conv2"]
    p3, p4 = params["conv3"], params["conv4"]
    Aext, B, xt = _tc_node(x_pad, p1)
    agg = _edge_phase(p1, Aext, B, src2, dst2, zrs)
    _, Aext, B, xt = _tc_combine_node(agg, xt, p1, p2)
    agg = _edge_phase(p2, Aext, B, src2, dst2, zrs)
    h2, Aext, B, xt = _tc_combine_node(agg, xt, p2, p3)
    x1 = _tc_pool(h2, b2, params["pool1"])
    agg = _edge_phase(p3, Aext, B, src2, dst2, zrs)
    _, Aext, B, xt = _tc_combine_node(agg, xt, p3, p4)
    agg = _edge_phase(p4, Aext, B, src2, dst2, zrs)
    h4 = _tc_combine(agg, xt, p4)
    x2 = _tc_pool(h4, b2, params["pool2"])
    return jnp.concatenate([x1, x2], axis=1)
